# baseline (speedup 1.0000x reference)
import functools

import jax
import jax.numpy as jnp
from jax import lax
from jax.experimental import pallas as pl
from jax.experimental.pallas import tpu as pltpu
from jax.experimental.pallas import tpu_sc as plsc

N = 100000
H = 128
G = 512
B = 1000
NB = N // B
NEG = -3.0e38

NW = 32
SC_C = 128
NCH = 25
N_PAD = NW * SC_C * NCH
ROWS_PER_SUB = G // 16


def _shiftL(v, d, pad):
    return jnp.concatenate(
        [v[:, d:], jnp.full((1, d), pad, v.dtype)], axis=1)


@functools.partial(
    pl.kernel,
    mesh=plsc.VectorSubcoreMesh(core_axis_name="c", subcore_axis_name="s"),
    out_type=jax.ShapeDtypeStruct((2 * G, H), jnp.float32),
    scratch_types=[
        pltpu.VMEM((SC_C, H), jnp.float32),
        pltpu.VMEM((SC_C,), jnp.int32),
        pltpu.VMEM((ROWS_PER_SUB, H), jnp.float32),
        pltpu.VMEM_SHARED((G, H), jnp.float32),
    ],
)
def _sc_segsum(x_hbm, seg_hbm, sums_out, xbuf, idxbuf, zbuf, acc):
    cid = lax.axis_index("c")
    sid = lax.axis_index("s")
    wid = cid * 16 + sid

    for r in range(ROWS_PER_SUB):
        for cb in range(H // 16):
            zbuf[r, pl.ds(cb * 16, 16)] = jnp.zeros((16,), jnp.float32)
    row0 = pl.multiple_of(sid * ROWS_PER_SUB, 8)
    pltpu.sync_copy(zbuf, acc.at[pl.ds(row0, ROWS_PER_SUB)])
    plsc.subcore_barrier()

    base = pl.multiple_of(wid * (SC_C * NCH), 8)
    for ch in range(NCH):
        off = pl.multiple_of(base + ch * SC_C, 8)
        pltpu.sync_copy(x_hbm.at[pl.ds(off, SC_C)], xbuf)
        pltpu.sync_copy(seg_hbm.at[pl.ds(off, SC_C)], idxbuf)
        pltpu.sync_copy(xbuf, acc.at[idxbuf], add=True)
    plsc.subcore_barrier()

    out0 = pl.multiple_of(cid * G + sid * ROWS_PER_SUB, 8)
    pltpu.sync_copy(acc.at[pl.ds(row0, ROWS_PER_SUB)],
                    sums_out.at[pl.ds(out0, ROWS_PER_SUB)])


def _tc_main(x_ref, seg_ref, wg1_ref, bg1_ref, wg2_ref,
             maxs_ref, cnts_ref, esum_ref, exsum_ref, rmax):
    i = pl.program_id(0)
    x = x_ref[...]
    xb = x.astype(jnp.bfloat16)
    seg_row = seg_ref[0]

    h = jnp.maximum(
        jnp.dot(xb, wg1_ref[...], preferred_element_type=jnp.float32)
        + bg1_ref[...], 0.0)
    gate = jnp.dot(h.astype(jnp.bfloat16), wg2_ref[...],
                   preferred_element_type=jnp.float32)

    tail_row = seg_row != _shiftL(seg_row, 1, -1)
    seg_col = seg_row.reshape(B, 1)

    m = jnp.concatenate([xb, gate.astype(jnp.bfloat16)], axis=1)
    d = 1
    while d < B:
        seg_sh = jnp.concatenate(
            [jnp.full((d, 1), -1, jnp.int32), seg_col[:-d, :]], axis=0)
        ok_col = seg_sh == seg_col
        m_sh = jnp.concatenate(
            [jnp.full((d, H + 1), NEG, jnp.bfloat16), m[:-d, :]], axis=0)
        m = jnp.maximum(m, jnp.where(ok_col, m_sh, jnp.bfloat16(NEG)))
        d *= 2

    iota_g = jax.lax.broadcasted_iota(jnp.int32, (G, 1), 0)
    oh = (iota_g == seg_row).astype(jnp.bfloat16)
    oh_tail = jnp.where(tail_row, oh, jnp.bfloat16(0))

    ones_col = jnp.ones((B, 1), jnp.bfloat16)
    rhs_tail = jnp.concatenate([m, ones_col], axis=1)
    tl = jax.lax.dot_general(oh_tail, rhs_tail, (((1,), (0,)), ((), ())),
                             preferred_element_type=jnp.float32)
    present = tl[:, H + 1:H + 2] > 0
    mx_blk = jnp.where(present, tl[:, :H], NEG)
    gmx_blk = jnp.where(present, tl[:, H:H + 1], NEG)

    gathered = jax.lax.dot_general(
        oh, gmx_blk.astype(jnp.bfloat16), (((0,), (0,)), ((), ())),
        preferred_element_type=jnp.float32)
    e = jnp.exp(gate - gathered)

    e_col = e.astype(jnp.bfloat16)
    y = xb * e_col
    rhs_big = jnp.concatenate([y, ones_col, e_col], axis=1)
    big = jax.lax.dot_general(oh, rhs_big, (((1,), (0,)), ((), ())),
                              preferred_element_type=jnp.float32)
    ex_blk = big[:, :H]
    c_blk = big[:, H:H + 1]
    es_blk = big[:, H + 1:H + 2]

    @pl.when(i == 0)
    def _():
        maxs_ref[...] = mx_blk
        cnts_ref[...] = c_blk
        rmax[...] = gmx_blk
        esum_ref[...] = es_blk
        exsum_ref[...] = ex_blk

    @pl.when(i > 0)
    def _():
        maxs_ref[...] = jnp.maximum(maxs_ref[...], mx_blk)
        cnts_ref[...] += c_blk
        r_old = rmax[...]
        r_new = jnp.maximum(r_old, gmx_blk)
        scale_old = jnp.exp(r_old - r_new)
        scale_blk = jnp.exp(gmx_blk - r_new)
        esum_ref[...] = esum_ref[...] * scale_old + es_blk * scale_blk
        exsum_ref[...] = exsum_ref[...] * scale_old + ex_blk * scale_blk
        rmax[...] = r_new


def _tc_final(sums2_ref, maxs_ref, cnts_ref, esum_ref, exsum_ref,
              wpa_ref, wpb_ref, wpc_ref, bp_ref, gamma_ref, beta_ref,
              out_ref):
    sums = sums2_ref[:G, :] + sums2_ref[G:, :]
    cnt = cnts_ref[...]
    nonempty = cnt > 0
    z_mean = sums / jnp.maximum(cnt, 1.0)
    z_max = jnp.where(nonempty, maxs_ref[...], float('-inf'))
    z_attn = exsum_ref[...] / jnp.maximum(esum_ref[...], 1e-30)
    z = (jnp.dot(z_mean, wpa_ref[...], preferred_element_type=jnp.float32)
         + jnp.dot(z_max, wpb_ref[...], preferred_element_type=jnp.float32)
         + jnp.dot(z_attn, wpc_ref[...], preferred_element_type=jnp.float32)
         + bp_ref[...])
    mu = jnp.mean(z, axis=1, keepdims=True)
    var = jnp.mean((z - mu) ** 2, axis=1, keepdims=True)
    out_ref[...] = ((z - mu) * jax.lax.rsqrt(var + 1e-5) * gamma_ref[...]
                    + beta_ref[...])


def kernel(x, batch, W_g1, b_g1, W_g2, b_g2, W_p, b_p, gamma, beta):
    seg_i = batch.astype(jnp.int32)
    seg = seg_i.reshape(NB, 1, B)
    bg1 = b_g1.reshape(1, H // 4)

    xp = jnp.concatenate([x, jnp.zeros((N_PAD - N, H), jnp.float32)], axis=0)
    segp = jnp.concatenate(
        [seg_i, jnp.full((N_PAD - N,), G - 1, jnp.int32)], axis=0)
    sums2 = _sc_segsum(xp, segp)

    full = lambda shp: pl.BlockSpec(shp, lambda i: tuple(0 for _ in shp))
    maxs, cnts, esum, exsum = pl.pallas_call(
        _tc_main,
        grid=(NB,),
        in_specs=[
            pl.BlockSpec((B, H), lambda i: (i, 0)),
            pl.BlockSpec((1, 1, B), lambda i: (i, 0, 0)),
            full((H, H // 4)),
            full((1, H // 4)),
            full((H // 4, 1)),
        ],
        out_specs=[full((G, H)), full((G, 1)), full((G, 1)), full((G, H))],
        out_shape=[
            jax.ShapeDtypeStruct((G, H), jnp.float32),
            jax.ShapeDtypeStruct((G, 1), jnp.float32),
            jax.ShapeDtypeStruct((G, 1), jnp.float32),
            jax.ShapeDtypeStruct((G, H), jnp.float32),
        ],
        scratch_shapes=[pltpu.VMEM((G, 1), jnp.float32)],
        compiler_params=pltpu.CompilerParams(
            dimension_semantics=("arbitrary",)),
    )(x, seg, W_g1.astype(jnp.bfloat16), bg1, W_g2.astype(jnp.bfloat16))

    out = pl.pallas_call(
        _tc_final,
        out_shape=jax.ShapeDtypeStruct((G, H), jnp.float32),
    )(sums2, maxs, cnts, esum, exsum,
      W_p[:H], W_p[H:2 * H], W_p[2 * H:], b_p.reshape(1, H),
      gamma.reshape(1, H), beta.reshape(1, H))
    return out
